# baseline (speedup 1.0000x reference)
import functools

import jax
import jax.numpy as jnp
from jax import lax
from jax.experimental import pallas as pl
from jax.experimental.pallas import tpu as pltpu
from jax.experimental.pallas import tpu_sc as plsc

D = 128
B = 4096
L = 200

NC = 2
NS = 16
NW = NC * NS
RPW = B // NW
LANES = 16
NJ = D // LANES
C0, C1 = 104, 96

_mesh = plsc.VectorSubcoreMesh(core_axis_name="c", subcore_axis_name="s")


def _pool_body(body_hbm, table_hbm, out_hbm, idx_v, buf0, buf1, out_v, e0_v,
               sem0, sem1):
    wid = lax.axis_index("s") * NC + lax.axis_index("c")
    base = wid * RPW
    pltpu.sync_copy(body_hbm.at[pl.ds(base, RPW)], idx_v)
    pltpu.sync_copy(table_hbm.at[pl.ds(0, 1)], e0_v)

    def _issue(row, buf, sem):
        pltpu.async_copy(table_hbm.at[idx_v.at[row, pl.ds(0, C0)]],
                         buf.at[pl.ds(0, C0)], sem)
        pltpu.async_copy(table_hbm.at[idx_v.at[row, pl.ds(C0, C1)]],
                         buf.at[pl.ds(C0, C1)], sem)

    def _wait(buf, sem):
        pltpu.make_async_copy(table_hbm.at[idx_v.at[0, pl.ds(0, C0)]],
                              buf.at[pl.ds(0, C0)], sem).wait()
        pltpu.make_async_copy(table_hbm.at[idx_v.at[0, pl.ds(C0, C1)]],
                              buf.at[pl.ds(C0, C1)], sem).wait()

    lane = lax.iota(jnp.int32, LANES)
    e0 = [e0_v[0, pl.ds(j * LANES, LANES)] for j in range(NJ)]
    zero = jnp.zeros((LANES,), jnp.float32)

    def _process(row, buf):
        cnt = zero
        for c in range(12):
            cnt = cnt + jnp.where(idx_v[row, pl.ds(c * 16, 16)] > 0, 1.0, 0.0)
        tail = (idx_v[row, pl.ds(184, 16)] > 0) & (lane >= 8)
        cnt = cnt + jnp.where(tail, 1.0, 0.0)
        nnzf = jnp.broadcast_to(jnp.sum(cnt), (LANES,))
        n0 = float(L) - nnzf
        inv = 1.0 / jnp.maximum(nnzf, 1.0)

        def acc_step(t, accs):
            l = t * 4
            for u in range(4):
                accs = tuple(accs[j] + buf[l + u, pl.ds(j * LANES, LANES)]
                             for j in range(NJ))
            return accs

        accs = lax.fori_loop(0, L // 4, acc_step,
                             tuple(zero for _ in range(NJ)))
        for j in range(NJ):
            out_v[row, pl.ds(j * LANES, LANES)] = (accs[j] - n0 * e0[j]) * inv

    _issue(0, buf0, sem0)

    def outer(t, carry):
        r0 = 2 * t
        r1 = r0 + 1
        _issue(r1, buf1, sem1)
        _wait(buf0, sem0)
        _process(r0, buf0)

        @pl.when(r1 + 1 < RPW)
        def _():
            _issue(r1 + 1, buf0, sem0)

        _wait(buf1, sem1)
        _process(r1, buf1)
        return carry

    lax.fori_loop(0, RPW // 2, outer, 0)
    pltpu.sync_copy(out_v, out_hbm.at[pl.ds(base, RPW)])


_pool = functools.partial(
    pl.kernel,
    out_type=jax.ShapeDtypeStruct((B, D), jnp.float32),
    mesh=_mesh,
    scratch_types=[
        pltpu.VMEM((RPW, L), jnp.int32),
        pltpu.VMEM((L, D), jnp.float32),
        pltpu.VMEM((L, D), jnp.float32),
        pltpu.VMEM((RPW, D), jnp.float32),
        pltpu.VMEM((1, D), jnp.float32),
        pltpu.SemaphoreType.DMA,
        pltpu.SemaphoreType.DMA,
    ],
    compiler_params=pltpu.CompilerParams(use_tc_tiling_on_sc=False,
                                         needs_layout_passes=False),
)(_pool_body)


def _bn_body(x_ref, g_ref, b_ref, o_ref):
    x = x_ref[...]
    mu = jnp.mean(x, axis=0, keepdims=True)
    xc = x - mu
    var = jnp.mean(xc * xc, axis=0, keepdims=True)
    o_ref[...] = g_ref[...] * (xc * lax.rsqrt(var + 1e-5)) + b_ref[...]


_bn = pl.pallas_call(
    _bn_body,
    out_shape=jax.ShapeDtypeStruct((B, D), jnp.float32),
)


def kernel(title, body, emb_table, gamma, beta):
    del title
    pooled = _pool(body.astype(jnp.int32), emb_table)
    return _bn(pooled, gamma.reshape(1, D), beta.reshape(1, D))


# speedup vs baseline: 1.2235x; 1.2235x over previous
import functools

import jax
import jax.numpy as jnp
from jax import lax
from jax.experimental import pallas as pl
from jax.experimental.pallas import tpu as pltpu
from jax.experimental.pallas import tpu_sc as plsc

D = 128
B = 4096
L = 200

NC = 2
NS = 16
NW = NC * NS
RPW = B // NW
LANES = 16
NJ = D // LANES
C0, C1 = 104, 96

_mesh = plsc.VectorSubcoreMesh(core_axis_name="c", subcore_axis_name="s")


def _pool_body(body_hbm, table_hbm, out_hbm, idx_v, buf0, buf1, buf2, out_v,
               e0_v, sem0, sem1, sem2):
    wid = lax.axis_index("s") * NC + lax.axis_index("c")
    base = wid * RPW
    pltpu.sync_copy(body_hbm.at[pl.ds(base, RPW)], idx_v)
    pltpu.sync_copy(table_hbm.at[pl.ds(0, 1)], e0_v)

    def _issue(row, buf, sem):
        pltpu.async_copy(table_hbm.at[idx_v.at[row, pl.ds(0, C0)]],
                         buf.at[pl.ds(0, C0)], sem)
        pltpu.async_copy(table_hbm.at[idx_v.at[row, pl.ds(C0, C1)]],
                         buf.at[pl.ds(C0, C1)], sem)

    def _wait(buf, sem):
        pltpu.make_async_copy(table_hbm.at[idx_v.at[0, pl.ds(0, C0)]],
                              buf.at[pl.ds(0, C0)], sem).wait()
        pltpu.make_async_copy(table_hbm.at[idx_v.at[0, pl.ds(C0, C1)]],
                              buf.at[pl.ds(C0, C1)], sem).wait()

    lane = lax.iota(jnp.int32, LANES)
    e0 = [e0_v[0, pl.ds(j * LANES, LANES)] for j in range(NJ)]
    zero = jnp.zeros((LANES,), jnp.float32)

    def _process(row, buf):
        cnt = zero
        for c in range(12):
            cnt = cnt + jnp.where(idx_v[row, pl.ds(c * 16, 16)] > 0, 1.0, 0.0)
        tail = (idx_v[row, pl.ds(184, 16)] > 0) & (lane >= 8)
        cnt = cnt + jnp.where(tail, 1.0, 0.0)
        nnzf = jnp.broadcast_to(jnp.sum(cnt), (LANES,))
        n0 = float(L) - nnzf
        inv = 1.0 / jnp.maximum(nnzf, 1.0)

        def acc_step(t, accs):
            l = t * 4
            for u in range(4):
                accs = tuple(accs[j] + buf[l + u, pl.ds(j * LANES, LANES)]
                             for j in range(NJ))
            return accs

        accs = lax.fori_loop(0, L // 4, acc_step,
                             tuple(zero for _ in range(NJ)))
        for j in range(NJ):
            out_v[row, pl.ds(j * LANES, LANES)] = (accs[j] - n0 * e0[j]) * inv

    bufs = (buf0, buf1, buf2)
    sems = (sem0, sem1, sem2)
    _issue(0, buf0, sem0)
    _issue(1, buf1, sem1)
    _issue(2, buf2, sem2)

    def outer(t, carry):
        for b in range(3):
            row = 3 * t + b
            _wait(bufs[b], sems[b])
            _process(row, bufs[b])

            @pl.when(row + 3 < RPW)
            def _():
                _issue(row + 3, bufs[b], sems[b])

        return carry

    lax.fori_loop(0, RPW // 3, outer, 0)
    for b in range(RPW % 3):
        row = (RPW // 3) * 3 + b
        _wait(bufs[b], sems[b])
        _process(row, bufs[b])
    pltpu.sync_copy(out_v, out_hbm.at[pl.ds(base, RPW)])


_pool = functools.partial(
    pl.kernel,
    out_type=jax.ShapeDtypeStruct((B, D), jnp.float32),
    mesh=_mesh,
    scratch_types=[
        pltpu.VMEM((RPW, L), jnp.int32),
        pltpu.VMEM((L, D), jnp.float32),
        pltpu.VMEM((L, D), jnp.float32),
        pltpu.VMEM((L, D), jnp.float32),
        pltpu.VMEM((RPW, D), jnp.float32),
        pltpu.VMEM((1, D), jnp.float32),
        pltpu.SemaphoreType.DMA,
        pltpu.SemaphoreType.DMA,
        pltpu.SemaphoreType.DMA,
    ],
    compiler_params=pltpu.CompilerParams(use_tc_tiling_on_sc=False,
                                         needs_layout_passes=False),
)(_pool_body)


def _bn_body(x_ref, g_ref, b_ref, o_ref):
    x = x_ref[...]
    mu = jnp.mean(x, axis=0, keepdims=True)
    xc = x - mu
    var = jnp.mean(xc * xc, axis=0, keepdims=True)
    o_ref[...] = g_ref[...] * (xc * lax.rsqrt(var + 1e-5)) + b_ref[...]


_bn = pl.pallas_call(
    _bn_body,
    out_shape=jax.ShapeDtypeStruct((B, D), jnp.float32),
)


def kernel(title, body, emb_table, gamma, beta):
    del title
    pooled = _pool(body.astype(jnp.int32), emb_table)
    return _bn(pooled, gamma.reshape(1, D), beta.reshape(1, D))
